# initial kernel scaffold (unmeasured)
import jax
import jax.numpy as jnp
from jax import lax
from jax.experimental import pallas as pl
from jax.experimental.pallas import tpu as pltpu

N_DEV = 4
E = 32
CAP = 204
TILE = 512


def _ag_route(route_shard):
    tok = route_shard.shape[0]

    def body(r_ref, out_ref, comm, send_sems, recv_sems):
        my = lax.axis_index("i")
        left = lax.rem(my + N_DEV - 1, N_DEV)
        right = lax.rem(my + 1, N_DEV)

        barrier = pltpu.get_barrier_semaphore()
        for nbr in (left, right):
            pl.semaphore_signal(barrier, inc=1, device_id=(nbr,),
                                device_id_type=pl.DeviceIdType.MESH)
        pl.semaphore_wait(barrier, 2)

        out_ref[pl.ds(my, 1)] = r_ref[...][None]
        for h in range(N_DEV - 1):
            src = r_ref if h == 0 else comm.at[h - 1]
            rdma = pltpu.make_async_remote_copy(
                src_ref=src,
                dst_ref=comm.at[h],
                send_sem=send_sems.at[h],
                recv_sem=recv_sems.at[h],
                device_id=(right,),
                device_id_type=pl.DeviceIdType.MESH,
            )
            rdma.start()
            rdma.wait()
            origin = lax.rem(my + 2 * N_DEV - 1 - h, N_DEV)
            out_ref[pl.ds(origin, 1)] = comm[h][None]

        def _exit(second_barrier):
            for nbr in (left, right):
                pl.semaphore_signal(second_barrier, inc=1, device_id=(nbr,),
                                    device_id_type=pl.DeviceIdType.MESH)
            pl.semaphore_wait(second_barrier, 2)

        pl.run_scoped(_exit, second_barrier=pltpu.SemaphoreType.REGULAR)

    return pl.pallas_call(
        body,
        out_shape=jax.ShapeDtypeStruct((N_DEV, tok, 1), jnp.int32),
        in_specs=[pl.BlockSpec(memory_space=pltpu.VMEM)],
        out_specs=pl.BlockSpec(memory_space=pltpu.VMEM),
        scratch_shapes=[
            pltpu.VMEM((N_DEV - 1, tok, 1), jnp.int32),
            pltpu.SemaphoreType.DMA((N_DEV - 1,)),
            pltpu.SemaphoreType.DMA((N_DEV - 1,)),
        ],
        compiler_params=pltpu.CompilerParams(collective_id=0),
    )(route_shard)


def _keep_masks(routeg, my):
    n_tok = routeg.shape[0] * routeg.shape[1]
    r = routeg.reshape(n_tok)
    oh = (r[:, None] == jnp.arange(E, dtype=r.dtype)[None, :]).astype(jnp.float32)
    g = oh.reshape(64, n_tok // 64, E)
    w = g.shape[1]
    m_in = (jnp.arange(w)[:, None] >= jnp.arange(w)[None, :]).astype(jnp.float32)
    pref = jnp.einsum("ij,gje->gie", m_in, g,
                      preferred_element_type=jnp.float32)
    tot = pref[:, -1, :]
    m_ex = (jnp.arange(64)[:, None] > jnp.arange(64)[None, :]).astype(jnp.float32)
    gpre = jnp.dot(m_ex, tot, preferred_element_type=jnp.float32)
    rank_excl = pref - g + gpre[:, None, :]
    kept = (g > 0.5) & (rank_excl < jnp.float32(CAP))
    keptf = kept.astype(jnp.float32).reshape(n_tok, E)

    e_loc = E // N_DEV
    mine = lax.dynamic_slice(keptf, (0, my * e_loc), (n_tok, e_loc))
    blocks = mine.reshape(N_DEV, n_tok // N_DEV, e_loc)
    return jnp.roll(blocks[::-1], my, axis=0)


def _moe_main(x, expert_w, kept):
    tok, d = x.shape
    e_loc, _, h_dim = expert_w.shape
    n_tiles = tok // TILE
    rs_slot = [0, 1, 0]

    def body(x_ref, w_ref, k_ref, out_ref, xg, rs,
             xsend, xrecv, rssend, rsrecv, credit):
        my = lax.axis_index("i")
        left = lax.rem(my + N_DEV - 1, N_DEV)
        right = lax.rem(my + 1, N_DEV)

        barrier = pltpu.get_barrier_semaphore()
        for nbr in (left, right):
            pl.semaphore_signal(barrier, inc=1, device_id=(nbr,),
                                device_id_type=pl.DeviceIdType.MESH)
        pl.semaphore_wait(barrier, 2)

        for hop in range(N_DEV - 1):
            src = x_ref if hop == 0 else xg.at[hop - 1]
            rdma = pltpu.make_async_remote_copy(
                src_ref=src,
                dst_ref=xg.at[hop],
                send_sem=xsend.at[hop],
                recv_sem=xrecv.at[hop],
                device_id=(right,),
                device_id_type=pl.DeviceIdType.MESH,
            )
            rdma.start()
            rdma.wait()

        def accum_block(slot, base_slot):
            for ti in range(n_tiles):
                r0 = ti * TILE
                if base_slot is None:
                    acc = jnp.zeros((TILE, h_dim), jnp.float32)
                else:
                    acc = rs[base_slot, r0:r0 + TILE, :]
                if slot == N_DEV - 1:
                    xt = x_ref[r0:r0 + TILE, :]
                else:
                    xt = xg[slot, r0:r0 + TILE, :]
                for e in range(e_loc):
                    ke = k_ref[slot, r0:r0 + TILE, e:e + 1]
                    acc = acc + jnp.dot(xt * ke, w_ref[e],
                                        preferred_element_type=jnp.float32)
                out_ref[r0:r0 + TILE, :] = acc

        for t in range(N_DEV - 1):
            accum_block(slot=t, base_slot=None if t == 0 else rs_slot[t - 1])
            if t == 1:
                pl.semaphore_signal(credit, inc=1, device_id=(left,),
                                    device_id_type=pl.DeviceIdType.MESH)
            if t == 2:
                pl.semaphore_wait(credit, 1)
            rdma = pltpu.make_async_remote_copy(
                src_ref=out_ref,
                dst_ref=rs.at[rs_slot[t]],
                send_sem=rssend.at[t],
                recv_sem=rsrecv.at[rs_slot[t]],
                device_id=(right,),
                device_id_type=pl.DeviceIdType.MESH,
            )
            rdma.start()
            rdma.wait()

        accum_block(slot=N_DEV - 1, base_slot=rs_slot[N_DEV - 2])

    return pl.pallas_call(
        body,
        out_shape=jax.ShapeDtypeStruct((tok, h_dim), jnp.float32),
        in_specs=[pl.BlockSpec(memory_space=pltpu.VMEM)] * 3,
        out_specs=pl.BlockSpec(memory_space=pltpu.VMEM),
        scratch_shapes=[
            pltpu.VMEM((N_DEV - 1, tok, d), jnp.float32),
            pltpu.VMEM((2, tok, h_dim), jnp.float32),
            pltpu.SemaphoreType.DMA((N_DEV - 1,)),
            pltpu.SemaphoreType.DMA((N_DEV - 1,)),
            pltpu.SemaphoreType.DMA((N_DEV - 1,)),
            pltpu.SemaphoreType.DMA((2,)),
            pltpu.SemaphoreType.REGULAR,
        ],
        compiler_params=pltpu.CompilerParams(collective_id=1),
    )(x, expert_w, kept)


def kernel(x, router_W, route_idx, expert_W):
    del router_W
    my = lax.axis_index("i")
    routeg = _ag_route(route_idx)
    kept = _keep_masks(routeg, my)
    return _moe_main(x, expert_W, kept)


# baseline (device time: 557471 ns/iter reference)
import jax
import jax.numpy as jnp
from jax import lax
from jax.experimental import pallas as pl
from jax.experimental.pallas import tpu as pltpu

N_DEV = 4
E = 32
CAP = 204
TILE = 512


def _ag_route(route_shard):
    tok = route_shard.shape[0]

    def body(r_ref, out_ref, comm, send_sems, recv_sems):
        my = lax.axis_index("i")
        left = lax.rem(my + N_DEV - 1, N_DEV)
        right = lax.rem(my + 1, N_DEV)

        barrier = pltpu.get_barrier_semaphore()
        for nbr in (left, right):
            pl.semaphore_signal(barrier, inc=1, device_id=(nbr,),
                                device_id_type=pl.DeviceIdType.MESH)
        pl.semaphore_wait(barrier, 2)

        out_ref[pl.ds(my, 1)] = r_ref[...][None]
        for h in range(N_DEV - 1):
            src = r_ref if h == 0 else comm.at[h - 1]
            rdma = pltpu.make_async_remote_copy(
                src_ref=src,
                dst_ref=comm.at[h],
                send_sem=send_sems.at[h],
                recv_sem=recv_sems.at[h],
                device_id=(right,),
                device_id_type=pl.DeviceIdType.MESH,
            )
            rdma.start()
            rdma.wait()
            origin = lax.rem(my + 2 * N_DEV - 1 - h, N_DEV)
            out_ref[pl.ds(origin, 1)] = comm[h][None]

        def _exit(second_barrier):
            for nbr in (left, right):
                pl.semaphore_signal(second_barrier, inc=1, device_id=(nbr,),
                                    device_id_type=pl.DeviceIdType.MESH)
            pl.semaphore_wait(second_barrier, 2)

        pl.run_scoped(_exit, second_barrier=pltpu.SemaphoreType.REGULAR)

    return pl.pallas_call(
        body,
        out_shape=jax.ShapeDtypeStruct((N_DEV, tok, 1), jnp.int32),
        in_specs=[pl.BlockSpec(memory_space=pltpu.VMEM)],
        out_specs=pl.BlockSpec(memory_space=pltpu.VMEM),
        scratch_shapes=[
            pltpu.VMEM((N_DEV - 1, tok, 1), jnp.int32),
            pltpu.SemaphoreType.DMA((N_DEV - 1,)),
            pltpu.SemaphoreType.DMA((N_DEV - 1,)),
        ],
        compiler_params=pltpu.CompilerParams(collective_id=0),
    )(route_shard)


def _keep_masks(routeg, my):
    n_tok = routeg.shape[0] * routeg.shape[1]
    r = routeg.reshape(n_tok)
    oh = (r[:, None] == jnp.arange(E, dtype=r.dtype)[None, :]).astype(jnp.float32)
    g = oh.reshape(64, n_tok // 64, E)
    w = g.shape[1]
    m_in = (jnp.arange(w)[:, None] >= jnp.arange(w)[None, :]).astype(jnp.float32)
    pref = jnp.einsum("ij,gje->gie", m_in, g,
                      preferred_element_type=jnp.float32)
    tot = pref[:, -1, :]
    m_ex = (jnp.arange(64)[:, None] > jnp.arange(64)[None, :]).astype(jnp.float32)
    gpre = jnp.dot(m_ex, tot, preferred_element_type=jnp.float32)
    rank_excl = pref - g + gpre[:, None, :]
    kept = (g > 0.5) & (rank_excl < jnp.float32(CAP))
    keptf = kept.astype(jnp.float32).reshape(n_tok, E)

    e_loc = E // N_DEV
    mine = lax.dynamic_slice(keptf, (0, my * e_loc), (n_tok, e_loc))
    blocks = mine.reshape(N_DEV, n_tok // N_DEV, e_loc)
    return jnp.roll(blocks[::-1], my, axis=0)


def _moe_main(x, expert_w, kept):
    tok, d = x.shape
    e_loc, _, h_dim = expert_w.shape
    n_tiles = tok // TILE
    rs_slot = [0, 1, 0]

    def body(x_ref, w_ref, k_ref, out_ref, xg, rs,
             xsend, xrecv, rssend, rsrecv, credit):
        my = lax.axis_index("i")
        left = lax.rem(my + N_DEV - 1, N_DEV)
        right = lax.rem(my + 1, N_DEV)

        barrier = pltpu.get_barrier_semaphore()
        for nbr in (left, right):
            pl.semaphore_signal(barrier, inc=1, device_id=(nbr,),
                                device_id_type=pl.DeviceIdType.MESH)
        pl.semaphore_wait(barrier, 2)

        for hop in range(N_DEV - 1):
            src = x_ref if hop == 0 else xg.at[hop - 1]
            rdma = pltpu.make_async_remote_copy(
                src_ref=src,
                dst_ref=xg.at[hop],
                send_sem=xsend.at[hop],
                recv_sem=xrecv.at[hop],
                device_id=(right,),
                device_id_type=pl.DeviceIdType.MESH,
            )
            rdma.start()
            rdma.wait()

        def accum_block(slot, base_slot):
            for ti in range(n_tiles):
                r0 = ti * TILE
                if base_slot is None:
                    acc = jnp.zeros((TILE, h_dim), jnp.float32)
                else:
                    acc = rs[base_slot, r0:r0 + TILE, :]
                if slot == N_DEV - 1:
                    xt = x_ref[r0:r0 + TILE, :]
                else:
                    xt = xg[slot, r0:r0 + TILE, :]
                for e in range(e_loc):
                    ke = k_ref[slot, r0:r0 + TILE, e:e + 1]
                    acc = acc + jnp.dot(xt * ke, w_ref[e],
                                        preferred_element_type=jnp.float32)
                out_ref[r0:r0 + TILE, :] = acc

        for t in range(N_DEV - 1):
            accum_block(slot=t, base_slot=None if t == 0 else rs_slot[t - 1])
            if t == 1:
                pl.semaphore_signal(credit, inc=1, device_id=(left,),
                                    device_id_type=pl.DeviceIdType.MESH)
            if t == 2:
                pl.semaphore_wait(credit, 1)
            rdma = pltpu.make_async_remote_copy(
                src_ref=out_ref,
                dst_ref=rs.at[rs_slot[t]],
                send_sem=rssend.at[t],
                recv_sem=rsrecv.at[rs_slot[t]],
                device_id=(right,),
                device_id_type=pl.DeviceIdType.MESH,
            )
            rdma.start()
            rdma.wait()

        accum_block(slot=N_DEV - 1, base_slot=rs_slot[N_DEV - 2])

    return pl.pallas_call(
        body,
        out_shape=jax.ShapeDtypeStruct((tok, h_dim), jnp.float32),
        in_specs=[pl.BlockSpec(memory_space=pltpu.VMEM)] * 3,
        out_specs=pl.BlockSpec(memory_space=pltpu.VMEM),
        scratch_shapes=[
            pltpu.VMEM((N_DEV - 1, tok, d), jnp.float32),
            pltpu.VMEM((2, tok, h_dim), jnp.float32),
            pltpu.SemaphoreType.DMA((N_DEV - 1,)),
            pltpu.SemaphoreType.DMA((N_DEV - 1,)),
            pltpu.SemaphoreType.DMA((N_DEV - 1,)),
            pltpu.SemaphoreType.DMA((2,)),
            pltpu.SemaphoreType.REGULAR,
        ],
        compiler_params=pltpu.CompilerParams(
            collective_id=1, vmem_limit_bytes=100 * 1024 * 1024),
    )(x, expert_w, kept)


def kernel(x, router_W, route_idx, expert_W):
    del router_W
    my = lax.axis_index("i")
    routeg = _ag_route(route_idx)
    kept = _keep_masks(routeg, my)
    return _moe_main(x, expert_W, kept)


# device time: 224841 ns/iter; 2.4794x vs baseline; 2.4794x over previous
import jax
import jax.numpy as jnp
from jax import lax
from jax.experimental import pallas as pl
from jax.experimental.pallas import tpu as pltpu

N_DEV = 4
E = 32
CAP = 204


def _ag_route(route_shard):
    tok = route_shard.shape[0]

    def body(r_ref, out_ref, comm, send_sems, recv_sems):
        my = lax.axis_index("i")
        left = lax.rem(my + N_DEV - 1, N_DEV)
        right = lax.rem(my + 1, N_DEV)

        barrier = pltpu.get_barrier_semaphore()
        for nbr in (left, right):
            pl.semaphore_signal(barrier, inc=1, device_id=(nbr,),
                                device_id_type=pl.DeviceIdType.MESH)
        pl.semaphore_wait(barrier, 2)

        out_ref[pl.ds(my, 1)] = r_ref[...][None]
        for h in range(N_DEV - 1):
            src = r_ref if h == 0 else comm.at[h - 1]
            rdma = pltpu.make_async_remote_copy(
                src_ref=src,
                dst_ref=comm.at[h],
                send_sem=send_sems.at[h],
                recv_sem=recv_sems.at[h],
                device_id=(right,),
                device_id_type=pl.DeviceIdType.MESH,
            )
            rdma.start()
            rdma.wait()
            origin = lax.rem(my + 2 * N_DEV - 1 - h, N_DEV)
            out_ref[pl.ds(origin, 1)] = comm[h][None]

        def _exit(second_barrier):
            for nbr in (left, right):
                pl.semaphore_signal(second_barrier, inc=1, device_id=(nbr,),
                                    device_id_type=pl.DeviceIdType.MESH)
            pl.semaphore_wait(second_barrier, 2)

        pl.run_scoped(_exit, second_barrier=pltpu.SemaphoreType.REGULAR)

    return pl.pallas_call(
        body,
        out_shape=jax.ShapeDtypeStruct((N_DEV, tok, 1), jnp.int32),
        in_specs=[pl.BlockSpec(memory_space=pltpu.VMEM)],
        out_specs=pl.BlockSpec(memory_space=pltpu.VMEM),
        scratch_shapes=[
            pltpu.VMEM((N_DEV - 1, tok, 1), jnp.int32),
            pltpu.SemaphoreType.DMA((N_DEV - 1,)),
            pltpu.SemaphoreType.DMA((N_DEV - 1,)),
        ],
        compiler_params=pltpu.CompilerParams(collective_id=0),
    )(route_shard)


def _keep_masks(routeg, my):
    n_tok = routeg.shape[0] * routeg.shape[1]
    r = routeg.reshape(n_tok)
    oh = (r[:, None] == jnp.arange(E, dtype=r.dtype)[None, :]).astype(jnp.float32)
    g = oh.reshape(64, n_tok // 64, E)
    w = g.shape[1]
    m_in = (jnp.arange(w)[:, None] >= jnp.arange(w)[None, :]).astype(jnp.float32)
    pref = jnp.einsum("ij,gje->gie", m_in, g,
                      preferred_element_type=jnp.float32)
    tot = pref[:, -1, :]
    m_ex = (jnp.arange(64)[:, None] > jnp.arange(64)[None, :]).astype(jnp.float32)
    gpre = jnp.dot(m_ex, tot, preferred_element_type=jnp.float32)
    rank_excl = pref - g + gpre[:, None, :]
    kept = (g > 0.5) & (rank_excl < jnp.float32(CAP))
    keptf = kept.astype(jnp.bfloat16).reshape(n_tok, E)

    e_loc = E // N_DEV
    mine = lax.dynamic_slice(keptf, (0, my * e_loc), (n_tok, e_loc))
    blocks = mine.reshape(N_DEV, n_tok // N_DEV, e_loc)
    return jnp.roll(blocks[::-1], my, axis=0)


def _moe_main(xb, wb, kept):
    tok, d = xb.shape
    e_loc, _, h_dim = wb.shape
    hl = h_dim // 2

    def body(x_ref, w_ref, k_ref, out_ref,
             xg, pst_dm2, pst_dp1_lo, pst_dm1_hi, accbf, rs_r, rs_l,
             ag_send, ag_recv, rss_r, rsr_r, rss_l, rsr_l,
             credit_r, credit_l):
        my = lax.axis_index("i")
        left = lax.rem(my + N_DEV - 1, N_DEV)
        right = lax.rem(my + 1, N_DEV)

        barrier = pltpu.get_barrier_semaphore()
        for nbr in (left, right):
            pl.semaphore_signal(barrier, inc=1, device_id=(nbr,),
                                device_id_type=pl.DeviceIdType.MESH)
        pl.semaphore_wait(barrier, 2)

        def rcopy(src, dst, ssem, rsem, dev):
            return pltpu.make_async_remote_copy(
                src_ref=src, dst_ref=dst, send_sem=ssem, recv_sem=rsem,
                device_id=(dev,), device_id_type=pl.DeviceIdType.MESH)

        def partial_cols(x_val, kslot, c0, c1):
            acc = jnp.zeros((tok, c1 - c0), jnp.float32)
            for e in range(e_loc):
                ke = k_ref[kslot, :, e:e + 1]
                acc = acc + jnp.dot(x_val * ke, w_ref[e, :, c0:c1],
                                    preferred_element_type=jnp.float32)
            return acc

        ag0 = rcopy(x_ref, xg.at[0], ag_send.at[0], ag_recv.at[0], right)
        ag0.start()
        agl = rcopy(x_ref, xg.at[2], ag_send.at[2], ag_recv.at[2], left)
        agl.start()

        ag0.wait_recv()
        ag1 = rcopy(xg.at[0], xg.at[1], ag_send.at[1], ag_recv.at[1], right)
        ag1.start()

        accbf[0] = partial_cols(xg[0], 0, 0, hl).astype(jnp.bfloat16)
        rsr0 = rcopy(accbf.at[0], rs_r.at[0], rss_r.at[0], rsr_r.at[0], right)
        rsr0.start()

        agl.wait_recv()
        accbf[1] = partial_cols(xg[2], 2, hl, h_dim).astype(jnp.bfloat16)
        rsl0 = rcopy(accbf.at[1], rs_l.at[0], rss_l.at[0], rsr_l.at[0], left)
        rsl0.start()

        pst_dp1_lo[...] = partial_cols(xg[2], 2, 0, hl).astype(jnp.bfloat16)
        pst_dm1_hi[...] = partial_cols(xg[0], 0, hl, h_dim).astype(jnp.bfloat16)
        ag1.wait_recv()
        pst_dm2[...] = partial_cols(xg[1], 1, 0, h_dim).astype(jnp.bfloat16)

        rsr0.wait_recv()
        rsr0.wait_send()
        accbf[0] = rs_r[0] + pst_dm2[:, 0:hl]
        pl.semaphore_signal(credit_r, inc=1, device_id=(left,),
                            device_id_type=pl.DeviceIdType.MESH)
        rsr1 = rcopy(accbf.at[0], rs_r.at[1], rss_r.at[1], rsr_r.at[1], right)
        rsr1.start()

        rsl0.wait_recv()
        rsl0.wait_send()
        accbf[1] = rs_l[0] + pst_dm2[:, hl:h_dim]
        pl.semaphore_signal(credit_l, inc=1, device_id=(right,),
                            device_id_type=pl.DeviceIdType.MESH)
        rsl1 = rcopy(accbf.at[1], rs_l.at[1], rss_l.at[1], rsr_l.at[1], left)
        rsl1.start()

        rsr1.wait_recv()
        rsr1.wait_send()
        pl.semaphore_wait(credit_r, 1)
        accbf[0] = rs_r[1] + pst_dp1_lo[...]
        rsr2 = rcopy(accbf.at[0], rs_r.at[0], rss_r.at[2], rsr_r.at[0], right)
        rsr2.start()

        rsl1.wait_recv()
        rsl1.wait_send()
        pl.semaphore_wait(credit_l, 1)
        accbf[1] = rs_l[1] + pst_dm1_hi[...]
        rsl2 = rcopy(accbf.at[1], rs_l.at[0], rss_l.at[2], rsr_l.at[0], left)
        rsl2.start()

        rsr2.wait_recv()
        out_ref[:, 0:hl] = rs_r[0].astype(jnp.float32) \
            + partial_cols(x_ref[...], N_DEV - 1, 0, hl)
        rsl2.wait_recv()
        out_ref[:, hl:h_dim] = rs_l[0].astype(jnp.float32) \
            + partial_cols(x_ref[...], N_DEV - 1, hl, h_dim)

        rsr2.wait_send()
        rsl2.wait_send()
        ag0.wait_send()
        ag1.wait_send()
        agl.wait_send()

    return pl.pallas_call(
        body,
        out_shape=jax.ShapeDtypeStruct((tok, h_dim), jnp.float32),
        in_specs=[pl.BlockSpec(memory_space=pltpu.VMEM)] * 3,
        out_specs=pl.BlockSpec(memory_space=pltpu.VMEM),
        scratch_shapes=[
            pltpu.VMEM((3, tok, d), jnp.bfloat16),
            pltpu.VMEM((tok, h_dim), jnp.bfloat16),
            pltpu.VMEM((tok, hl), jnp.bfloat16),
            pltpu.VMEM((tok, hl), jnp.bfloat16),
            pltpu.VMEM((2, tok, hl), jnp.bfloat16),
            pltpu.VMEM((2, tok, hl), jnp.bfloat16),
            pltpu.VMEM((2, tok, hl), jnp.bfloat16),
            pltpu.SemaphoreType.DMA((3,)),
            pltpu.SemaphoreType.DMA((3,)),
            pltpu.SemaphoreType.DMA((3,)),
            pltpu.SemaphoreType.DMA((2,)),
            pltpu.SemaphoreType.DMA((3,)),
            pltpu.SemaphoreType.DMA((2,)),
            pltpu.SemaphoreType.REGULAR,
            pltpu.SemaphoreType.REGULAR,
        ],
        compiler_params=pltpu.CompilerParams(
            collective_id=1, vmem_limit_bytes=100 * 1024 * 1024),
    )(xb, wb, kept)


def kernel(x, router_W, route_idx, expert_W):
    del router_W
    my = lax.axis_index("i")
    routeg = _ag_route(route_idx)
    kept = _keep_masks(routeg, my)
    xb = x.astype(jnp.bfloat16)
    wb = expert_W.astype(jnp.bfloat16)
    return _moe_main(xb, wb, kept)


# device time: 188332 ns/iter; 2.9600x vs baseline; 1.1939x over previous
import jax
import jax.numpy as jnp
from jax import lax
from jax.experimental import pallas as pl
from jax.experimental.pallas import tpu as pltpu

N_DEV = 4
E = 32
CAP = 204


def _ag_route(route_shard):
    tok = route_shard.shape[0]

    def body(r_ref, out_ref, comm, send_sems, recv_sems):
        my = lax.axis_index("i")
        left = lax.rem(my + N_DEV - 1, N_DEV)
        right = lax.rem(my + 1, N_DEV)
        opp = lax.rem(my + 2, N_DEV)

        barrier = pltpu.get_barrier_semaphore()
        for nbr in (left, right):
            pl.semaphore_signal(barrier, inc=1, device_id=(nbr,),
                                device_id_type=pl.DeviceIdType.MESH)
        pl.semaphore_wait(barrier, 2)

        rdmas = []
        for j, tgt in ((0, right), (1, left), (2, opp)):
            rdma = pltpu.make_async_remote_copy(
                src_ref=r_ref,
                dst_ref=comm.at[j],
                send_sem=send_sems.at[j],
                recv_sem=recv_sems.at[j],
                device_id=(tgt,),
                device_id_type=pl.DeviceIdType.MESH,
            )
            rdma.start()
            rdmas.append(rdma)

        out_ref[pl.ds(my, 1)] = r_ref[...][None]
        for j, org in ((0, left), (1, right), (2, opp)):
            rdmas[j].wait_recv()
            out_ref[pl.ds(org, 1)] = comm[j][None]
        for rdma in rdmas:
            rdma.wait_send()

    return pl.pallas_call(
        body,
        out_shape=jax.ShapeDtypeStruct((N_DEV, tok, 1), jnp.int32),
        in_specs=[pl.BlockSpec(memory_space=pltpu.VMEM)],
        out_specs=pl.BlockSpec(memory_space=pltpu.VMEM),
        scratch_shapes=[
            pltpu.VMEM((N_DEV - 1, tok, 1), jnp.int32),
            pltpu.SemaphoreType.DMA((N_DEV - 1,)),
            pltpu.SemaphoreType.DMA((N_DEV - 1,)),
        ],
        compiler_params=pltpu.CompilerParams(collective_id=0),
    )(route_shard)


def _keep_masks(routeg, my):
    n_tok = routeg.shape[0] * routeg.shape[1]
    r = routeg.reshape(n_tok)
    oh = (r[:, None] == jnp.arange(E, dtype=r.dtype)[None, :]).astype(jnp.float32)
    g = oh.reshape(64, n_tok // 64, E)
    w = g.shape[1]
    m_in = (jnp.arange(w)[:, None] >= jnp.arange(w)[None, :]).astype(jnp.float32)
    pref = jnp.einsum("ij,gje->gie", m_in, g,
                      preferred_element_type=jnp.float32)
    tot = pref[:, -1, :]
    m_ex = (jnp.arange(64)[:, None] > jnp.arange(64)[None, :]).astype(jnp.float32)
    gpre = jnp.dot(m_ex, tot, preferred_element_type=jnp.float32)
    rank_excl = pref - g + gpre[:, None, :]
    kept = (g > 0.5) & (rank_excl < jnp.float32(CAP))
    keptf = kept.astype(jnp.bfloat16).reshape(n_tok, E)

    e_loc = E // N_DEV
    mine = lax.dynamic_slice(keptf, (0, my * e_loc), (n_tok, e_loc))
    blocks = mine.reshape(N_DEV, n_tok // N_DEV, e_loc)
    return jnp.roll(blocks[::-1], my, axis=0)


def _moe_main(xb, wb, kept):
    tok, d = xb.shape
    e_loc, _, h_dim = wb.shape
    hl = h_dim // 2

    def body(x_ref, w_ref, k_ref, out_ref,
             xg, pst_dm2, pst_dp1_lo, pst_dm1_hi, pst_own, accbf, rs_r, rs_l,
             ag_send, ag_recv, rss_r, rsr_r, rss_l, rsr_l,
             credit_r, credit_l):
        my = lax.axis_index("i")
        left = lax.rem(my + N_DEV - 1, N_DEV)
        right = lax.rem(my + 1, N_DEV)

        barrier = pltpu.get_barrier_semaphore()
        for nbr in (left, right):
            pl.semaphore_signal(barrier, inc=1, device_id=(nbr,),
                                device_id_type=pl.DeviceIdType.MESH)
        pl.semaphore_wait(barrier, 2)

        def rcopy(src, dst, ssem, rsem, dev):
            return pltpu.make_async_remote_copy(
                src_ref=src, dst_ref=dst, send_sem=ssem, recv_sem=rsem,
                device_id=(dev,), device_id_type=pl.DeviceIdType.MESH)

        def partial_cols(x_val, kslot, c0, c1):
            acc = jnp.zeros((tok, c1 - c0), jnp.float32)
            for e in range(e_loc):
                ke = k_ref[kslot, :, e:e + 1]
                acc = acc + jnp.dot(x_val * ke, w_ref[e, :, c0:c1],
                                    preferred_element_type=jnp.float32)
            return acc

        ag0 = rcopy(x_ref, xg.at[0], ag_send.at[0], ag_recv.at[0], right)
        ag0.start()
        agl = rcopy(x_ref, xg.at[2], ag_send.at[2], ag_recv.at[2], left)
        agl.start()

        ag0.wait_recv()
        ag1 = rcopy(xg.at[0], xg.at[1], ag_send.at[1], ag_recv.at[1], right)
        ag1.start()

        accbf[0] = partial_cols(xg[0], 0, 0, hl).astype(jnp.bfloat16)
        rsr0 = rcopy(accbf.at[0], rs_r.at[0], rss_r.at[0], rsr_r.at[0], right)
        rsr0.start()

        agl.wait_recv()
        accbf[1] = partial_cols(xg[2], 2, hl, h_dim).astype(jnp.bfloat16)
        rsl0 = rcopy(accbf.at[1], rs_l.at[0], rss_l.at[0], rsr_l.at[0], left)
        rsl0.start()

        pst_own[:, 0:hl] = partial_cols(x_ref[...], N_DEV - 1, 0, hl) \
            .astype(jnp.bfloat16)

        ag1.wait_recv()
        pst_dm2[:, 0:hl] = partial_cols(xg[1], 1, 0, hl).astype(jnp.bfloat16)
        rsr0.wait_recv()
        rsr0.wait_send()
        accbf[0] = rs_r[0] + pst_dm2[:, 0:hl]
        pl.semaphore_signal(credit_r, inc=1, device_id=(left,),
                            device_id_type=pl.DeviceIdType.MESH)
        rsr1 = rcopy(accbf.at[0], rs_r.at[1], rss_r.at[1], rsr_r.at[1], right)
        rsr1.start()

        pst_dm2[:, hl:h_dim] = partial_cols(xg[1], 1, hl, h_dim) \
            .astype(jnp.bfloat16)
        rsl0.wait_recv()
        rsl0.wait_send()
        accbf[1] = rs_l[0] + pst_dm2[:, hl:h_dim]
        pl.semaphore_signal(credit_l, inc=1, device_id=(right,),
                            device_id_type=pl.DeviceIdType.MESH)
        rsl1 = rcopy(accbf.at[1], rs_l.at[1], rss_l.at[1], rsr_l.at[1], left)
        rsl1.start()

        pst_dp1_lo[...] = partial_cols(xg[2], 2, 0, hl).astype(jnp.bfloat16)
        rsr1.wait_recv()
        rsr1.wait_send()
        pl.semaphore_wait(credit_r, 1)
        accbf[0] = rs_r[1] + pst_dp1_lo[...]
        rsr2 = rcopy(accbf.at[0], rs_r.at[0], rss_r.at[2], rsr_r.at[0], right)
        rsr2.start()

        pst_dm1_hi[...] = partial_cols(xg[0], 0, hl, h_dim).astype(jnp.bfloat16)
        rsl1.wait_recv()
        rsl1.wait_send()
        pl.semaphore_wait(credit_l, 1)
        accbf[1] = rs_l[1] + pst_dm1_hi[...]
        rsl2 = rcopy(accbf.at[1], rs_l.at[0], rss_l.at[2], rsr_l.at[0], left)
        rsl2.start()

        pst_own[:, hl:h_dim] = partial_cols(x_ref[...], N_DEV - 1, hl, h_dim) \
            .astype(jnp.bfloat16)
        rsr2.wait_recv()
        out_ref[:, 0:hl] = (rs_r[0] + pst_own[:, 0:hl]).astype(jnp.float32)
        rsl2.wait_recv()
        out_ref[:, hl:h_dim] = (rs_l[0] + pst_own[:, hl:h_dim]) \
            .astype(jnp.float32)

        rsr2.wait_send()
        rsl2.wait_send()
        ag0.wait_send()
        ag1.wait_send()
        agl.wait_send()

    return pl.pallas_call(
        body,
        out_shape=jax.ShapeDtypeStruct((tok, h_dim), jnp.float32),
        in_specs=[pl.BlockSpec(memory_space=pltpu.VMEM)] * 3,
        out_specs=pl.BlockSpec(memory_space=pltpu.VMEM),
        scratch_shapes=[
            pltpu.VMEM((3, tok, d), jnp.bfloat16),
            pltpu.VMEM((tok, h_dim), jnp.bfloat16),
            pltpu.VMEM((tok, hl), jnp.bfloat16),
            pltpu.VMEM((tok, hl), jnp.bfloat16),
            pltpu.VMEM((tok, h_dim), jnp.bfloat16),
            pltpu.VMEM((2, tok, hl), jnp.bfloat16),
            pltpu.VMEM((2, tok, hl), jnp.bfloat16),
            pltpu.VMEM((2, tok, hl), jnp.bfloat16),
            pltpu.SemaphoreType.DMA((3,)),
            pltpu.SemaphoreType.DMA((3,)),
            pltpu.SemaphoreType.DMA((3,)),
            pltpu.SemaphoreType.DMA((2,)),
            pltpu.SemaphoreType.DMA((3,)),
            pltpu.SemaphoreType.DMA((2,)),
            pltpu.SemaphoreType.REGULAR,
            pltpu.SemaphoreType.REGULAR,
        ],
        compiler_params=pltpu.CompilerParams(
            collective_id=1, vmem_limit_bytes=100 * 1024 * 1024),
    )(xb, wb, kept)


def kernel(x, router_W, route_idx, expert_W):
    del router_W
    my = lax.axis_index("i")
    routeg = _ag_route(route_idx)
    kept = _keep_masks(routeg, my)
    xb = x.astype(jnp.bfloat16)
    wb = expert_W.astype(jnp.bfloat16)
    return _moe_main(xb, wb, kept)


# device time: 185689 ns/iter; 3.0022x vs baseline; 1.0142x over previous
import jax
import jax.numpy as jnp
from jax import lax
from jax.experimental import pallas as pl
from jax.experimental.pallas import tpu as pltpu

N_DEV = 4
E = 32
CAP = 204


def _ag_route(route_shard):
    tok = route_shard.shape[0]

    def body(r_ref, out_ref, comm, send_sems, recv_sems):
        my = lax.axis_index("i")
        left = lax.rem(my + N_DEV - 1, N_DEV)
        right = lax.rem(my + 1, N_DEV)
        opp = lax.rem(my + 2, N_DEV)

        barrier = pltpu.get_barrier_semaphore()
        for nbr in (left, right):
            pl.semaphore_signal(barrier, inc=1, device_id=(nbr,),
                                device_id_type=pl.DeviceIdType.MESH)
        pl.semaphore_wait(barrier, 2)

        rdmas = []
        for j, tgt in ((0, right), (1, left), (2, opp)):
            rdma = pltpu.make_async_remote_copy(
                src_ref=r_ref,
                dst_ref=comm.at[j],
                send_sem=send_sems.at[j],
                recv_sem=recv_sems.at[j],
                device_id=(tgt,),
                device_id_type=pl.DeviceIdType.MESH,
            )
            rdma.start()
            rdmas.append(rdma)

        out_ref[pl.ds(my, 1)] = r_ref[...][None]
        for j, org in ((0, left), (1, right), (2, opp)):
            rdmas[j].wait_recv()
            out_ref[pl.ds(org, 1)] = comm[j][None]
        for rdma in rdmas:
            rdma.wait_send()

    return pl.pallas_call(
        body,
        out_shape=jax.ShapeDtypeStruct((N_DEV, tok, 1), jnp.int32),
        in_specs=[pl.BlockSpec(memory_space=pltpu.VMEM)],
        out_specs=pl.BlockSpec(memory_space=pltpu.VMEM),
        scratch_shapes=[
            pltpu.VMEM((N_DEV - 1, tok, 1), jnp.int32),
            pltpu.SemaphoreType.DMA((N_DEV - 1,)),
            pltpu.SemaphoreType.DMA((N_DEV - 1,)),
        ],
        compiler_params=pltpu.CompilerParams(collective_id=0),
    )(route_shard)


def _keep_masks(routeg, my):
    n_tok = routeg.shape[0] * routeg.shape[1]
    r = routeg.reshape(n_tok)
    oh = (r[:, None] == jnp.arange(E, dtype=r.dtype)[None, :]).astype(jnp.float32)
    g = oh.reshape(64, n_tok // 64, E)
    w = g.shape[1]
    m_in = (jnp.arange(w)[:, None] >= jnp.arange(w)[None, :]).astype(jnp.float32)
    pref = jnp.einsum("ij,gje->gie", m_in, g,
                      preferred_element_type=jnp.float32)
    tot = pref[:, -1, :]
    m_ex = (jnp.arange(64)[:, None] > jnp.arange(64)[None, :]).astype(jnp.float32)
    gpre = jnp.dot(m_ex, tot, preferred_element_type=jnp.float32)
    rank_excl = pref - g + gpre[:, None, :]
    kept = (g > 0.5) & (rank_excl < jnp.float32(CAP))
    keptf = kept.astype(jnp.bfloat16).reshape(n_tok, E)

    e_loc = E // N_DEV
    mine = lax.dynamic_slice(keptf, (0, my * e_loc), (n_tok, e_loc))
    blocks = mine.reshape(N_DEV, n_tok // N_DEV, e_loc)
    return jnp.roll(blocks[::-1], my, axis=0)


def _moe_main(xb, wb, kept):
    tok, d = xb.shape
    e_loc, _, h_dim = wb.shape
    hl = h_dim // 2
    hh = tok // 2

    def body(x_ref, w_ref, k_ref, out_ref,
             xg, pst_dm2, pst_dp1_lo, pst_dm1_hi, pst_own, accbf, rs_r, rs_l,
             ag_send, ag_recv, rss_r, rsr_r, rss_l, rsr_l,
             credit_r, credit_l):
        my = lax.axis_index("i")
        left = lax.rem(my + N_DEV - 1, N_DEV)
        right = lax.rem(my + 1, N_DEV)

        barrier = pltpu.get_barrier_semaphore()
        for nbr in (left, right):
            pl.semaphore_signal(barrier, inc=1, device_id=(nbr,),
                                device_id_type=pl.DeviceIdType.MESH)
        pl.semaphore_wait(barrier, 2)

        def rcopy(src, dst, ssem, rsem, dev):
            return pltpu.make_async_remote_copy(
                src_ref=src, dst_ref=dst, send_sem=ssem, recv_sem=rsem,
                device_id=(dev,), device_id_type=pl.DeviceIdType.MESH)

        def partial_cols(x_val, kslot, c0, c1):
            acc = jnp.zeros((tok, c1 - c0), jnp.float32)
            for e in range(e_loc):
                ke = k_ref[kslot, :, e:e + 1]
                acc = acc + jnp.dot(x_val * ke, w_ref[e, :, c0:c1],
                                    preferred_element_type=jnp.float32)
            return acc

        ag0 = rcopy(x_ref, xg.at[0], ag_send.at[0], ag_recv.at[0], right)
        ag0.start()
        agl = rcopy(x_ref, xg.at[2], ag_send.at[2], ag_recv.at[2], left)
        agl.start()

        ag0.wait_recv()
        ag1 = rcopy(xg.at[0], xg.at[1], ag_send.at[1], ag_recv.at[1], right)
        ag1.start()

        def chain_pair(acc_slot, rs_buf, rs_slot, ssems, rsems, t, dev):
            return [rcopy(accbf.at[acc_slot, pl.ds(c * hh, hh)],
                          rs_buf.at[rs_slot, pl.ds(c * hh, hh)],
                          ssems.at[2 * t + c],
                          rsems.at[2 * rs_slot + c], dev)
                    for c in range(2)]

        accbf[0] = partial_cols(xg[0], 0, 0, hl).astype(jnp.bfloat16)
        rsr0 = chain_pair(0, rs_r, 0, rss_r, rsr_r, 0, right)
        for rd in rsr0:
            rd.start()

        agl.wait_recv()
        accbf[1] = partial_cols(xg[2], 2, hl, h_dim).astype(jnp.bfloat16)
        rsl0 = chain_pair(1, rs_l, 0, rss_l, rsr_l, 0, left)
        for rd in rsl0:
            rd.start()

        pst_own[:, 0:hl] = partial_cols(x_ref[...], N_DEV - 1, 0, hl) \
            .astype(jnp.bfloat16)

        def step(prev_pair, acc_slot, rs_buf, prev_slot, pst_val, ssems,
                 rsems, t, dev, credit=None):
            new_slot = prev_slot ^ 1 if t == 1 else 0
            cur = chain_pair(acc_slot, rs_buf, new_slot, ssems, rsems, t, dev)
            for c in range(2):
                r0 = c * hh
                prev_pair[c].wait_recv()
                prev_pair[c].wait_send()
                if c == 0 and credit is not None:
                    pl.semaphore_wait(credit, 1)
                accbf[acc_slot, r0:r0 + hh] = \
                    rs_buf[prev_slot, r0:r0 + hh] + pst_val[r0:r0 + hh]
                cur[c].start()
            return cur

        ag1.wait_recv()
        pst_dm2[:, 0:hl] = partial_cols(xg[1], 1, 0, hl).astype(jnp.bfloat16)
        rsr1 = step(rsr0, 0, rs_r, 0, pst_dm2[:, 0:hl], rss_r, rsr_r,
                    1, right)
        pl.semaphore_signal(credit_r, inc=1, device_id=(left,),
                            device_id_type=pl.DeviceIdType.MESH)

        pst_dm2[:, hl:h_dim] = partial_cols(xg[1], 1, hl, h_dim) \
            .astype(jnp.bfloat16)
        rsl1 = step(rsl0, 1, rs_l, 0, pst_dm2[:, hl:h_dim], rss_l, rsr_l,
                    1, left)
        pl.semaphore_signal(credit_l, inc=1, device_id=(right,),
                            device_id_type=pl.DeviceIdType.MESH)

        pst_dp1_lo[...] = partial_cols(xg[2], 2, 0, hl).astype(jnp.bfloat16)
        rsr2 = step(rsr1, 0, rs_r, 1, pst_dp1_lo[...], rss_r, rsr_r,
                    2, right, credit=credit_r)

        pst_dm1_hi[...] = partial_cols(xg[0], 0, hl, h_dim).astype(jnp.bfloat16)
        rsl2 = step(rsl1, 1, rs_l, 1, pst_dm1_hi[...], rss_l, rsr_l,
                    2, left, credit=credit_l)

        pst_own[:, hl:h_dim] = partial_cols(x_ref[...], N_DEV - 1, hl, h_dim) \
            .astype(jnp.bfloat16)
        for c in range(2):
            r0 = c * hh
            rsr2[c].wait_recv()
            out_ref[r0:r0 + hh, 0:hl] = \
                (rs_r[0, r0:r0 + hh] + pst_own[r0:r0 + hh, 0:hl]) \
                .astype(jnp.float32)
        for c in range(2):
            r0 = c * hh
            rsl2[c].wait_recv()
            out_ref[r0:r0 + hh, hl:h_dim] = \
                (rs_l[0, r0:r0 + hh] + pst_own[r0:r0 + hh, hl:h_dim]) \
                .astype(jnp.float32)

        for rd in rsr2 + rsl2:
            rd.wait_send()
        ag0.wait_send()
        ag1.wait_send()
        agl.wait_send()

    return pl.pallas_call(
        body,
        out_shape=jax.ShapeDtypeStruct((tok, h_dim), jnp.float32),
        in_specs=[pl.BlockSpec(memory_space=pltpu.VMEM)] * 3,
        out_specs=pl.BlockSpec(memory_space=pltpu.VMEM),
        scratch_shapes=[
            pltpu.VMEM((3, tok, d), jnp.bfloat16),
            pltpu.VMEM((tok, h_dim), jnp.bfloat16),
            pltpu.VMEM((tok, hl), jnp.bfloat16),
            pltpu.VMEM((tok, hl), jnp.bfloat16),
            pltpu.VMEM((tok, h_dim), jnp.bfloat16),
            pltpu.VMEM((2, tok, hl), jnp.bfloat16),
            pltpu.VMEM((2, tok, hl), jnp.bfloat16),
            pltpu.VMEM((2, tok, hl), jnp.bfloat16),
            pltpu.SemaphoreType.DMA((3,)),
            pltpu.SemaphoreType.DMA((3,)),
            pltpu.SemaphoreType.DMA((6,)),
            pltpu.SemaphoreType.DMA((4,)),
            pltpu.SemaphoreType.DMA((6,)),
            pltpu.SemaphoreType.DMA((4,)),
            pltpu.SemaphoreType.REGULAR,
            pltpu.SemaphoreType.REGULAR,
        ],
        compiler_params=pltpu.CompilerParams(
            collective_id=1, vmem_limit_bytes=100 * 1024 * 1024),
    )(xb, wb, kept)


def kernel(x, router_W, route_idx, expert_W):
    del router_W
    my = lax.axis_index("i")
    routeg = _ag_route(route_idx)
    kept = _keep_masks(routeg, my)
    xb = x.astype(jnp.bfloat16)
    wb = expert_W.astype(jnp.bfloat16)
    return _moe_main(xb, wb, kept)


# device time: 185146 ns/iter; 3.0110x vs baseline; 1.0029x over previous
import jax
import jax.numpy as jnp
from jax import lax
from jax.experimental import pallas as pl
from jax.experimental.pallas import tpu as pltpu

N_DEV = 4
E = 32
CAP = 204


def _ag_route(route_shard):
    tok = route_shard.shape[0]

    def body(r_ref, out_ref, comm, send_sems, recv_sems):
        my = lax.axis_index("i")
        left = lax.rem(my + N_DEV - 1, N_DEV)
        right = lax.rem(my + 1, N_DEV)
        opp = lax.rem(my + 2, N_DEV)

        barrier = pltpu.get_barrier_semaphore()
        for nbr in (left, right):
            pl.semaphore_signal(barrier, inc=1, device_id=(nbr,),
                                device_id_type=pl.DeviceIdType.MESH)
        pl.semaphore_wait(barrier, 2)

        rdmas = []
        for j, tgt in ((0, right), (1, left), (2, opp)):
            rdma = pltpu.make_async_remote_copy(
                src_ref=r_ref,
                dst_ref=comm.at[j],
                send_sem=send_sems.at[j],
                recv_sem=recv_sems.at[j],
                device_id=(tgt,),
                device_id_type=pl.DeviceIdType.MESH,
            )
            rdma.start()
            rdmas.append(rdma)

        out_ref[pl.ds(my, 1)] = r_ref[...][None]
        for j, org in ((0, left), (1, right), (2, opp)):
            rdmas[j].wait_recv()
            out_ref[pl.ds(org, 1)] = comm[j][None]
        for rdma in rdmas:
            rdma.wait_send()

    return pl.pallas_call(
        body,
        out_shape=jax.ShapeDtypeStruct((N_DEV, tok, 1), jnp.int32),
        in_specs=[pl.BlockSpec(memory_space=pltpu.VMEM)],
        out_specs=pl.BlockSpec(memory_space=pltpu.VMEM),
        scratch_shapes=[
            pltpu.VMEM((N_DEV - 1, tok, 1), jnp.int32),
            pltpu.SemaphoreType.DMA((N_DEV - 1,)),
            pltpu.SemaphoreType.DMA((N_DEV - 1,)),
        ],
        compiler_params=pltpu.CompilerParams(collective_id=0),
    )(route_shard)


def _keep_masks(routeg, my):
    n_tok = routeg.shape[0] * routeg.shape[1]
    e_loc = E // N_DEV
    r = routeg.reshape(n_tok)
    my_experts = my * e_loc + jnp.arange(e_loc, dtype=r.dtype)
    oh = (r[:, None] == my_experts[None, :]).astype(jnp.float32)
    g = oh.reshape(64, n_tok // 64, e_loc)
    w = g.shape[1]
    m_in = (jnp.arange(w)[:, None] >= jnp.arange(w)[None, :]).astype(jnp.float32)
    pref = jnp.einsum("ij,gje->gie", m_in, g,
                      preferred_element_type=jnp.float32)
    tot = pref[:, -1, :]
    m_ex = (jnp.arange(64)[:, None] > jnp.arange(64)[None, :]).astype(jnp.float32)
    gpre = jnp.dot(m_ex, tot, preferred_element_type=jnp.float32)
    rank_excl = pref - g + gpre[:, None, :]
    kept = (g > 0.5) & (rank_excl < jnp.float32(CAP))
    blocks = kept.astype(jnp.bfloat16).reshape(N_DEV, n_tok // N_DEV, e_loc)
    return jnp.roll(blocks[::-1], my, axis=0)


def _moe_main(xb, wb, kept):
    tok, d = xb.shape
    e_loc, _, h_dim = wb.shape
    hl = h_dim // 2
    hh = tok // 2

    def body(x_ref, w_ref, k_ref, out_ref,
             xg, pst_dm2, pst_dp1_lo, pst_dm1_hi, pst_own, accbf, rs_r, rs_l,
             ag_send, ag_recv, rss_r, rsr_r, rss_l, rsr_l,
             credit_r, credit_l):
        my = lax.axis_index("i")
        left = lax.rem(my + N_DEV - 1, N_DEV)
        right = lax.rem(my + 1, N_DEV)

        barrier = pltpu.get_barrier_semaphore()
        for nbr in (left, right):
            pl.semaphore_signal(barrier, inc=1, device_id=(nbr,),
                                device_id_type=pl.DeviceIdType.MESH)
        pl.semaphore_wait(barrier, 2)

        def rcopy(src, dst, ssem, rsem, dev):
            return pltpu.make_async_remote_copy(
                src_ref=src, dst_ref=dst, send_sem=ssem, recv_sem=rsem,
                device_id=(dev,), device_id_type=pl.DeviceIdType.MESH)

        def partial_cols(x_val, kslot, c0, c1):
            acc = jnp.zeros((tok, c1 - c0), jnp.float32)
            for e in range(e_loc):
                ke = k_ref[kslot, :, e:e + 1]
                acc = acc + jnp.dot(x_val * ke, w_ref[e, :, c0:c1],
                                    preferred_element_type=jnp.float32)
            return acc

        ag0 = rcopy(x_ref, xg.at[0], ag_send.at[0], ag_recv.at[0], right)
        ag0.start()
        agl = rcopy(x_ref, xg.at[2], ag_send.at[2], ag_recv.at[2], left)
        agl.start()

        ag0.wait_recv()
        ag1 = rcopy(xg.at[0], xg.at[1], ag_send.at[1], ag_recv.at[1], right)
        ag1.start()

        def chain_pair(acc_slot, rs_buf, rs_slot, ssems, rsems, t, dev):
            return [rcopy(accbf.at[acc_slot, pl.ds(c * hh, hh)],
                          rs_buf.at[rs_slot, pl.ds(c * hh, hh)],
                          ssems.at[2 * t + c],
                          rsems.at[2 * rs_slot + c], dev)
                    for c in range(2)]

        accbf[0] = partial_cols(xg[0], 0, 0, hl).astype(jnp.bfloat16)
        rsr0 = chain_pair(0, rs_r, 0, rss_r, rsr_r, 0, right)
        for rd in rsr0:
            rd.start()

        agl.wait_recv()
        accbf[1] = partial_cols(xg[2], 2, hl, h_dim).astype(jnp.bfloat16)
        rsl0 = chain_pair(1, rs_l, 0, rss_l, rsr_l, 0, left)
        for rd in rsl0:
            rd.start()

        pst_own[:, 0:hl] = partial_cols(x_ref[...], N_DEV - 1, 0, hl) \
            .astype(jnp.bfloat16)

        def step(prev_pair, acc_slot, rs_buf, prev_slot, pst_val, ssems,
                 rsems, t, dev, credit=None):
            new_slot = prev_slot ^ 1 if t == 1 else 0
            cur = chain_pair(acc_slot, rs_buf, new_slot, ssems, rsems, t, dev)
            for c in range(2):
                r0 = c * hh
                prev_pair[c].wait_recv()
                prev_pair[c].wait_send()
                if c == 0 and credit is not None:
                    pl.semaphore_wait(credit, 1)
                accbf[acc_slot, r0:r0 + hh] = \
                    rs_buf[prev_slot, r0:r0 + hh] + pst_val[r0:r0 + hh]
                cur[c].start()
            return cur

        ag1.wait_recv()
        pst_dm2[:, 0:hl] = partial_cols(xg[1], 1, 0, hl).astype(jnp.bfloat16)
        rsr1 = step(rsr0, 0, rs_r, 0, pst_dm2[:, 0:hl], rss_r, rsr_r,
                    1, right)
        pl.semaphore_signal(credit_r, inc=1, device_id=(left,),
                            device_id_type=pl.DeviceIdType.MESH)

        pst_dm2[:, hl:h_dim] = partial_cols(xg[1], 1, hl, h_dim) \
            .astype(jnp.bfloat16)
        rsl1 = step(rsl0, 1, rs_l, 0, pst_dm2[:, hl:h_dim], rss_l, rsr_l,
                    1, left)
        pl.semaphore_signal(credit_l, inc=1, device_id=(right,),
                            device_id_type=pl.DeviceIdType.MESH)

        pst_dp1_lo[...] = partial_cols(xg[2], 2, 0, hl).astype(jnp.bfloat16)
        rsr2 = step(rsr1, 0, rs_r, 1, pst_dp1_lo[...], rss_r, rsr_r,
                    2, right, credit=credit_r)

        pst_dm1_hi[...] = partial_cols(xg[0], 0, hl, h_dim).astype(jnp.bfloat16)
        rsl2 = step(rsl1, 1, rs_l, 1, pst_dm1_hi[...], rss_l, rsr_l,
                    2, left, credit=credit_l)

        pst_own[:, hl:h_dim] = partial_cols(x_ref[...], N_DEV - 1, hl, h_dim) \
            .astype(jnp.bfloat16)
        for c in range(2):
            r0 = c * hh
            rsr2[c].wait_recv()
            out_ref[r0:r0 + hh, 0:hl] = \
                (rs_r[0, r0:r0 + hh] + pst_own[r0:r0 + hh, 0:hl]) \
                .astype(jnp.float32)
        for c in range(2):
            r0 = c * hh
            rsl2[c].wait_recv()
            out_ref[r0:r0 + hh, hl:h_dim] = \
                (rs_l[0, r0:r0 + hh] + pst_own[r0:r0 + hh, hl:h_dim]) \
                .astype(jnp.float32)

        for rd in rsr2 + rsl2:
            rd.wait_send()
        ag0.wait_send()
        ag1.wait_send()
        agl.wait_send()

    return pl.pallas_call(
        body,
        out_shape=jax.ShapeDtypeStruct((tok, h_dim), jnp.float32),
        in_specs=[pl.BlockSpec(memory_space=pltpu.VMEM)] * 3,
        out_specs=pl.BlockSpec(memory_space=pltpu.VMEM),
        scratch_shapes=[
            pltpu.VMEM((3, tok, d), jnp.bfloat16),
            pltpu.VMEM((tok, h_dim), jnp.bfloat16),
            pltpu.VMEM((tok, hl), jnp.bfloat16),
            pltpu.VMEM((tok, hl), jnp.bfloat16),
            pltpu.VMEM((tok, h_dim), jnp.bfloat16),
            pltpu.VMEM((2, tok, hl), jnp.bfloat16),
            pltpu.VMEM((2, tok, hl), jnp.bfloat16),
            pltpu.VMEM((2, tok, hl), jnp.bfloat16),
            pltpu.SemaphoreType.DMA((3,)),
            pltpu.SemaphoreType.DMA((3,)),
            pltpu.SemaphoreType.DMA((6,)),
            pltpu.SemaphoreType.DMA((4,)),
            pltpu.SemaphoreType.DMA((6,)),
            pltpu.SemaphoreType.DMA((4,)),
            pltpu.SemaphoreType.REGULAR,
            pltpu.SemaphoreType.REGULAR,
        ],
        compiler_params=pltpu.CompilerParams(
            collective_id=1, vmem_limit_bytes=100 * 1024 * 1024),
    )(xb, wb, kept)


def kernel(x, router_W, route_idx, expert_W):
    del router_W
    my = lax.axis_index("i")
    routeg = _ag_route(route_idx)
    kept = _keep_masks(routeg, my)
    xb = x.astype(jnp.bfloat16)
    wb = expert_W.astype(jnp.bfloat16)
    return _moe_main(xb, wb, kept)
